# baseline (speedup 1.0000x reference)
import functools

import jax
import jax.numpy as jnp
from jax import lax
from jax.experimental import pallas as pl
from jax.experimental.pallas import tpu as pltpu
from jax.experimental.pallas import tpu_sc as plsc

N_NODES = 100000
FEAT = 128
NW = 32
CH = 16
ROWS = 7 * CH
HROWS = ROWS // 2
CPW_LO = 194
IDX_CAP = 200 * ROWS
NO_PAD = 7 * 96896 + IDX_CAP

_mesh = plsc.VectorSubcoreMesh(core_axis_name="c", subcore_axis_name="s")


@functools.partial(
    pl.kernel,
    mesh=_mesh,
    compiler_params=pltpu.CompilerParams(needs_layout_passes=False),
    out_type=[
        jax.ShapeDtypeStruct((N_NODES, FEAT), jnp.float32),
        jax.ShapeDtypeStruct((N_NODES, FEAT), jnp.int32),
    ],
    scratch_types=[
        pltpu.VMEM((IDX_CAP,), jnp.int32),
        pltpu.VMEM((ROWS, FEAT), jnp.float32),
        pltpu.VMEM((ROWS, FEAT), jnp.float32),
        pltpu.VMEM((CH, FEAT), jnp.float32),
        pltpu.VMEM((CH, FEAT), jnp.float32),
        pltpu.VMEM((CH, FEAT), jnp.int32),
        pltpu.VMEM((CH, FEAT), jnp.int32),
        pltpu.SemaphoreType.DMA,
        pltpu.SemaphoreType.DMA,
        pltpu.SemaphoreType.DMA,
        pltpu.SemaphoreType.DMA,
    ],
)
def _sc_pool(x_hbm, no_hbm, vals_hbm, idxs_hbm,
             idx_all, rows0, rows1, vout0, vout1, iout0, iout1,
             sem_g0, sem_g1, sem_o0, sem_o1):
    wid = lax.axis_index("s") * 2 + lax.axis_index("c")
    node0 = CH * CPW_LO * wid + 2 * CH * jnp.minimum(wid, 21)
    n_pairs = jnp.where(wid < 21, (CPW_LO + 2) // 2, CPW_LO // 2)

    iota = lax.iota(jnp.int32, 16)
    iota7 = iota * 7
    kf = [jnp.full((16,), float(k), jnp.float32) for k in range(7)]
    zeros16 = jnp.zeros((16,), jnp.int32)

    pltpu.sync_copy(no_hbm.at[pl.ds(node0 * 7, IDX_CAP)], idx_all)

    def gather(g, rows_ref, sem):
        base = g * ROWS
        for h in range(2):
            pltpu.async_copy(
                x_hbm.at[idx_all.at[pl.ds(base + h * HROWS, HROWS)]],
                rows_ref.at[pl.ds(h * HROWS, HROWS)], sem)

    def wait_gather(rows_ref, sem):
        for h in range(2):
            pltpu.make_async_copy(
                x_hbm.at[idx_all.at[pl.ds(0, HROWS)]],
                rows_ref.at[pl.ds(h * HROWS, HROWS)], sem).wait()

    def put_out(g, vout, iout, sem):
        node_base = node0 + g * CH
        pltpu.async_copy(vout, vals_hbm.at[pl.ds(node_base, CH)], sem)
        pltpu.async_copy(iout, idxs_hbm.at[pl.ds(node_base, CH)], sem)

    def wait_out(vout, iout, sem):
        pltpu.make_async_copy(vout, vals_hbm.at[pl.ds(0, CH)], sem).wait()
        pltpu.make_async_copy(iout, idxs_hbm.at[pl.ds(0, CH)], sem).wait()

    def compute(rows_ref, vout, iout):
        def pick(mv, mi, nv, ni):
            m = nv > mv
            return jnp.maximum(mv, nv), jnp.where(m, ni, mi)

        def node_body(b, _):
            base = b * 896
            for v in range(8):
                col = iota7 + (base + 112 * v)
                g = [plsc.load_gather(rows_ref,
                                      [zeros16, col + k if k else col])
                     for k in range(7)]
                v01, i01 = pick(g[0], kf[0], g[1], kf[1])
                v23, i23 = pick(g[2], kf[2], g[3], kf[3])
                v45, i45 = pick(g[4], kf[4], g[5], kf[5])
                v03, i03 = pick(v01, i01, v23, i23)
                v46, i46 = pick(v45, i45, g[6], kf[6])
                bval, bidx = pick(v03, i03, v46, i46)
                vout[b, pl.ds(16 * v, 16)] = bval
                iout[b, pl.ds(16 * v, 16)] = bidx.astype(jnp.int32)
            return 0

        lax.fori_loop(0, CH, node_body, 0)

    gather(0, rows0, sem_g0)

    def pair_body(m, _):
        g0 = 2 * m
        gather(g0 + 1, rows1, sem_g1)
        wait_gather(rows0, sem_g0)

        @pl.when(m > 0)
        def _():
            wait_out(vout0, iout0, sem_o0)

        compute(rows0, vout0, iout0)
        put_out(g0, vout0, iout0, sem_o0)
        gather(g0 + 2, rows0, sem_g0)

        wait_gather(rows1, sem_g1)

        @pl.when(m > 0)
        def _():
            wait_out(vout1, iout1, sem_o1)

        compute(rows1, vout1, iout1)
        put_out(g0 + 1, vout1, iout1, sem_o1)
        return 0

    lax.fori_loop(0, n_pairs, pair_body, 0)

    wait_gather(rows0, sem_g0)
    wait_out(vout0, iout0, sem_o0)
    wait_out(vout1, iout1, sem_o1)


def kernel(x, neigh_orders):
    no32 = neigh_orders.astype(jnp.int32)
    no32 = jnp.concatenate(
        [no32, jnp.zeros((NO_PAD - no32.shape[0],), jnp.int32)])
    vals, idxs = _sc_pool(x, no32)
    return (vals, idxs)


# speedup vs baseline: 1.0279x; 1.0279x over previous
import functools

import jax
import jax.numpy as jnp
from jax import lax
from jax.experimental import pallas as pl
from jax.experimental.pallas import tpu as pltpu
from jax.experimental.pallas import tpu_sc as plsc

N_NODES = 100000
FEAT = 128
NW = 32
CH = 32
ROWS = 7 * CH
QROWS = ROWS // 4
NSLOT = 2
CPW_LO = 97
IDX_CAP = 100 * ROWS
NO_PAD = 7 * 96896 + IDX_CAP

_mesh = plsc.VectorSubcoreMesh(core_axis_name="c", subcore_axis_name="s")


@functools.partial(
    pl.kernel,
    mesh=_mesh,
    compiler_params=pltpu.CompilerParams(needs_layout_passes=False),
    out_type=[
        jax.ShapeDtypeStruct((N_NODES, FEAT), jnp.float32),
        jax.ShapeDtypeStruct((N_NODES, FEAT), jnp.int32),
    ],
    scratch_types=[
        pltpu.VMEM((IDX_CAP,), jnp.int32),
        pltpu.VMEM((NSLOT * ROWS, FEAT), jnp.float32),
        pltpu.VMEM((NSLOT * CH, FEAT), jnp.float32),
        pltpu.VMEM((NSLOT * CH, FEAT), jnp.int32),
        pltpu.SemaphoreType.DMA((NSLOT,)),
        pltpu.SemaphoreType.DMA((NSLOT,)),
    ],
)
def _sc_pool(x_hbm, no_hbm, vals_hbm, idxs_hbm,
             idx_all, rows_all, vout_all, iout_all, sem_g, sem_o):
    wid = lax.axis_index("s") * 2 + lax.axis_index("c")
    node0 = CH * CPW_LO * wid + CH * jnp.minimum(wid, 21)
    n_chunks = jnp.where(wid < 21, CPW_LO + 1, CPW_LO)

    iota = lax.iota(jnp.int32, 16)
    iota7 = iota * 7
    kf = [jnp.full((16,), float(k), jnp.float32) for k in range(7)]
    zeros16 = jnp.zeros((16,), jnp.int32)

    pltpu.sync_copy(no_hbm.at[pl.ds(node0 * 7, IDX_CAP)], idx_all)

    def gather(g, slot):
        base = g * ROWS
        rbase = slot * ROWS
        for h in range(4):
            pltpu.async_copy(
                x_hbm.at[idx_all.at[pl.ds(base + h * QROWS, QROWS)]],
                rows_all.at[pl.ds(rbase + h * QROWS, QROWS)],
                sem_g.at[slot])

    def wait_gather(slot):
        for h in range(4):
            pltpu.make_async_copy(
                x_hbm.at[idx_all.at[pl.ds(0, QROWS)]],
                rows_all.at[pl.ds(h * QROWS, QROWS)],
                sem_g.at[slot]).wait()

    def put_out(g, slot):
        node_base = node0 + g * CH
        obase = slot * CH
        pltpu.async_copy(vout_all.at[pl.ds(obase, CH)],
                         vals_hbm.at[pl.ds(node_base, CH)], sem_o.at[slot])
        pltpu.async_copy(iout_all.at[pl.ds(obase, CH)],
                         idxs_hbm.at[pl.ds(node_base, CH)], sem_o.at[slot])

    def wait_out(slot):
        pltpu.make_async_copy(vout_all.at[pl.ds(0, CH)],
                              vals_hbm.at[pl.ds(0, CH)], sem_o.at[slot]).wait()
        pltpu.make_async_copy(iout_all.at[pl.ds(0, CH)],
                              idxs_hbm.at[pl.ds(0, CH)], sem_o.at[slot]).wait()

    def compute(slot):
        rbase = slot * ROWS
        obase = slot * CH

        def node_body(b, _):
            base = b * 896 + rbase * FEAT
            orow = b + obase
            for v in range(8):
                bval = None
                bidx = None
                for k in range(7):
                    col = iota7 + (base + (112 * v + k))
                    gv = plsc.load_gather(rows_all, [zeros16, col])
                    if k == 0:
                        bval = gv
                        bidx = kf[0]
                    else:
                        m = gv > bval
                        bval = jnp.maximum(bval, gv)
                        bidx = jnp.where(m, kf[k], bidx)
                vout_all[orow, pl.ds(16 * v, 16)] = bval
                iout_all[orow, pl.ds(16 * v, 16)] = bidx.astype(jnp.int32)
            return 0

        lax.fori_loop(0, CH, node_body, 0)

    for i in range(NSLOT):
        gather(i, i)

    def chunk_body(g, _):
        slot = g & (NSLOT - 1)
        wait_gather(slot)

        @pl.when(g >= NSLOT)
        def _():
            wait_out(slot)

        compute(slot)
        put_out(g, slot)
        gather(g + NSLOT, slot)
        return 0

    lax.fori_loop(0, n_chunks, chunk_body, 0)

    for i in range(NSLOT):
        wait_gather(i)
        wait_out(i)


def kernel(x, neigh_orders):
    no32 = neigh_orders.astype(jnp.int32)
    no32 = jnp.concatenate(
        [no32, jnp.zeros((NO_PAD - no32.shape[0],), jnp.int32)])
    vals, idxs = _sc_pool(x, no32)
    return (vals, idxs)
